# Initial kernel scaffold; baseline (speedup 1.0000x reference)
import jax
import jax.numpy as jnp
from jax.experimental import pallas as pl


def kernel(x_domain, atom_constants, predicate_ids, constant_table, predicate_table, W_kge, b_kge):
    raise NotImplementedError("write your pallas kernel here")


# speedup vs baseline: 3.2973x; 3.2973x over previous
import functools

import jax
import jax.numpy as jnp
from jax import lax
from jax.experimental import pallas as pl
from jax.experimental.pallas import tpu as pltpu
from jax.experimental.pallas import tpu_sc as plsc

_NC, _NS, _L = 2, 16, 16
_NW = _NC * _NS


def _sc_gather(x_domain, ac_flat, table):
    nq = x_domain.shape[0]
    f = ac_flat.shape[0]
    d = table.shape[1]
    per_w = f // _NW
    sub = 512
    n_sub = per_w // sub

    mesh = plsc.VectorSubcoreMesh(core_axis_name="c", subcore_axis_name="s")

    @functools.partial(
        pl.kernel,
        out_type=jax.ShapeDtypeStruct((f, d), jnp.float32),
        mesh=mesh,
        scratch_types=[
            pltpu.VMEM((per_w,), jnp.int32),
            pltpu.VMEM((per_w,), jnp.int32),
            pltpu.VMEM((sub, d), jnp.float32),
            pltpu.SemaphoreType.DMA,
        ],
        compiler_params=pltpu.CompilerParams(use_tc_tiling_on_sc=False),
    )
    def k(xd_hbm, ac_hbm, table_hbm, out_hbm, ac_v, idx_v, rows_v, sem):
        wid = lax.axis_index("s") * _NC + lax.axis_index("c")
        base = wid * per_w
        pltpu.sync_copy(ac_hbm.at[pl.ds(base, per_w)], ac_v)
        pltpu.async_copy(xd_hbm.at[ac_v], idx_v, sem).wait()

        @pl.loop(0, n_sub)
        def _rows(s):
            off = s * sub
            pltpu.async_copy(
                table_hbm.at[idx_v.at[pl.ds(off, sub)]], rows_v, sem
            ).wait()
            pltpu.sync_copy(rows_v, out_hbm.at[pl.ds(base + off, sub)])

    return k(x_domain, ac_flat, table)


def _tc_kge(cc, pid2, ptab, w, b2):
    n, dcc = cc.shape
    npred, pemb = ptab.shape
    dout = w.shape[1]
    blk = 2048
    grid = n // blk

    def body(cc_ref, pid_ref, ptab_ref, w_ref, b_ref, out_ref):
        pp = jnp.dot(ptab_ref[...], w_ref[0:pemb, :],
                     preferred_element_type=jnp.float32)
        oh = (pid_ref[...] == lax.broadcasted_iota(jnp.int32, (blk, npred), 1)
              ).astype(jnp.float32)
        acc = jnp.dot(oh, pp, preferred_element_type=jnp.float32)
        acc = acc + jnp.dot(cc_ref[...], w_ref[pemb:pemb + dcc, :],
                            preferred_element_type=jnp.float32)
        out_ref[...] = jnp.tanh(acc + b_ref[...])

    return pl.pallas_call(
        body,
        grid=(grid,),
        in_specs=[
            pl.BlockSpec((blk, dcc), lambda i: (i, 0)),
            pl.BlockSpec((blk, 1), lambda i: (i, 0)),
            pl.BlockSpec((npred, pemb), lambda i: (0, 0)),
            pl.BlockSpec(w.shape, lambda i: (0, 0)),
            pl.BlockSpec((1, dout), lambda i: (0, 0)),
        ],
        out_specs=pl.BlockSpec((blk, dout), lambda i: (i, 0)),
        out_shape=jax.ShapeDtypeStruct((n, dout), jnp.float32),
    )(cc, pid2, ptab, w, b2)


def kernel(x_domain, atom_constants, predicate_ids, constant_table,
           predicate_table, W_kge, b_kge):
    n_atoms = atom_constants.shape[0]
    ac_flat = atom_constants.astype(jnp.int32).reshape(-1)
    cc = _sc_gather(x_domain.astype(jnp.int32), ac_flat, constant_table)
    cc = cc.reshape(n_atoms, -1)
    pid2 = predicate_ids.astype(jnp.int32).reshape(-1, 1)
    return _tc_kge(cc, pid2, predicate_table, W_kge, b_kge.reshape(1, -1))
